# baseline (speedup 1.0000x reference)
import functools

import jax
import jax.numpy as jnp
from jax import lax
from jax.experimental import pallas as pl
from jax.experimental.pallas import tpu as pltpu
from jax.experimental.pallas import tpu_sc as plsc

VOCAB = 1000000
D = 128
SEQ = 200
BATCH = 1024
N = BATCH * SEQ
NC, NS = 2, 16
NW = NC * NS
PER_W = N // NW
CHUNK = 128
NCH = PER_W // CHUNK
POS_EXT = SEQ + CHUNK
LANES = 16
DV = D // LANES

_mesh = plsc.VectorSubcoreMesh(core_axis_name="c", subcore_axis_name="s")


@functools.partial(
    pl.kernel,
    out_type=jax.ShapeDtypeStruct((N, D), jnp.float32),
    mesh=_mesh,
    scratch_types=[
        pltpu.VMEM((POS_EXT, D), jnp.float32),
        pltpu.VMEM((CHUNK,), jnp.int32),
        pltpu.VMEM((CHUNK, D), jnp.float32),
        pltpu.SemaphoreType.DMA,
    ],
)
def _embed(seq_hbm, table_hbm, pos_hbm, out_hbm, pos_v, idx_v, rows_v, sem):
    wid = lax.axis_index("s") * NC + lax.axis_index("c")
    base = wid * PER_W

    pltpu.sync_copy(pos_hbm, pos_v.at[pl.ds(0, SEQ)])
    pltpu.sync_copy(pos_hbm.at[pl.ds(0, CHUNK)], pos_v.at[pl.ds(SEQ, CHUNK)])

    def chunk_body(c, carry):
        row0 = base + c * CHUNK
        pltpu.sync_copy(seq_hbm.at[pl.ds(row0, CHUNK)], idx_v)
        pltpu.async_copy(table_hbm.at[idx_v], rows_v, sem).wait()
        p0 = lax.rem(row0, SEQ)

        def add_body(i, carry2):
            pr = p0 + i
            for j in range(DV):
                s = pl.ds(j * LANES, LANES)
                rows_v[i, s] = rows_v[i, s] + pos_v[pr, s]
            return carry2

        lax.fori_loop(0, CHUNK, add_body, 0, unroll=2)
        pltpu.sync_copy(rows_v, out_hbm.at[pl.ds(row0, CHUNK)])
        return carry

    lax.fori_loop(0, NCH, chunk_body, 0)


def kernel(sequence, src_word_table, src_pos_table):
    out = _embed(sequence.reshape(N), src_word_table, src_pos_table)
    return out.reshape(BATCH, SEQ, D)


# speedup vs baseline: 3.5904x; 3.5904x over previous
import functools

import jax
import jax.numpy as jnp
from jax import lax
from jax.experimental import pallas as pl
from jax.experimental.pallas import tpu as pltpu
from jax.experimental.pallas import tpu_sc as plsc

VOCAB = 1000000
D = 128
SEQ = 200
BATCH = 1024
N = BATCH * SEQ
NC, NS = 2, 16
NW = NC * NS
PER_W = N // NW
CHUNK = 128
NCH = PER_W // CHUNK
POS_EXT = 320
LANES = 16
DV = D // LANES
NBUF = 4

_mesh = plsc.VectorSubcoreMesh(core_axis_name="c", subcore_axis_name="s")


@functools.partial(
    pl.kernel,
    out_type=jax.ShapeDtypeStruct((N, D), jnp.float32),
    mesh=_mesh,
    scratch_types=[
        pltpu.VMEM((POS_EXT, D), jnp.float32),
        pltpu.VMEM((NCH, CHUNK), jnp.int32),
        [pltpu.VMEM((CHUNK, D), jnp.float32)] * NBUF,
        [pltpu.SemaphoreType.DMA] * NBUF,
        [pltpu.SemaphoreType.DMA] * NBUF,
    ],
)
def _embed(seq_hbm, table_hbm, pos_hbm, out_hbm, pos_v, idx_v, rows, gsem, osem):
    wid = lax.axis_index("s") * NC + lax.axis_index("c")
    base = wid * PER_W

    def gather_start(c, b):
        pltpu.async_copy(table_hbm.at[idx_v.at[c]], rows[b], gsem[b])

    def gather_wait(b):
        pltpu.make_async_copy(table_hbm.at[pl.ds(0, CHUNK)], rows[b],
                              gsem[b]).wait()

    def out_start(c, b):
        pltpu.async_copy(rows[b], out_hbm.at[pl.ds(base + c * CHUNK, CHUNK)],
                         osem[b])

    def out_wait(b):
        pltpu.make_async_copy(rows[b], out_hbm.at[pl.ds(0, CHUNK)],
                              osem[b]).wait()

    def add_pos(c, b):
        p0 = lax.rem(c * CHUNK, SEQ)
        r = rows[b]

        @plsc.parallel_loop(0, CHUNK, unroll=4)
        def _(i):
            pr = p0 + i
            for j in range(DV):
                s = pl.ds(j * LANES, LANES)
                plsc.addupdate(r.at[i, s], pos_v[pr, s])

    def step(c, b, wait_out):
        gather_wait(b)
        add_pos(c, b)
        out_start(c, b)
        b2 = (b + 2) % NBUF
        if wait_out:
            out_wait(b2)
        gather_start(c + 2, b2)

    pltpu.sync_copy(seq_hbm.at[wid], idx_v)
    gather_start(0, 0)
    gather_start(1, 1)
    pltpu.sync_copy(pos_hbm, pos_v.at[pl.ds(0, SEQ)])
    pltpu.sync_copy(pos_hbm.at[pl.ds(0, CHUNK - DV)],
                    pos_v.at[pl.ds(SEQ, CHUNK - DV)])

    step(0, 0, False)
    step(1, 1, False)
    step(2, 2, True)
    step(3, 3, True)

    def super_body(s, carry):
        c0 = s * NBUF
        for b in range(NBUF):
            step(c0 + b, b, True)
        return carry

    lax.fori_loop(1, NCH // NBUF, super_body, 0)

    for b, c in ((0, NCH - 2), (1, NCH - 1)):
        gather_wait(b)
        add_pos(c, b)
        out_start(c, b)
        out_wait((b + 2) % NBUF)
    out_wait(0)
    out_wait(1)


def kernel(sequence, src_word_table, src_pos_table):
    out = _embed(sequence.reshape(NW, NCH, CHUNK), src_word_table,
                 src_pos_table)
    return out.reshape(BATCH, SEQ, D)
